# baseline (speedup 1.0000x reference)
import functools

import jax
import jax.numpy as jnp
from jax import lax
from jax.experimental import pallas as pl
from jax.experimental.pallas import tpu as pltpu
from jax.experimental.pallas import tpu_sc as plsc

_G = 8
_ROWS = 16384
_D = 128
_NC = 2
_NS = 16
_NW = _NC * _NS
_SC_ROWS = 5120
_RPW = _SC_ROWS // _NW
_CHUNK = 40
_NCHUNK = _RPW // _CHUNK
_PR = 2 * _G + 1
_PPAD = 24
_UNROLL = 8
_TCB = 512
_TC_ROWS = _ROWS - _SC_ROWS

_SPLAT_DNUMS = lax.GatherDimensionNumbers(
    offset_dims=(), collapsed_slice_dims=(0,), start_index_map=(0,))


def _splat(vec, r):
    idx = jnp.full((16, 1), r, jnp.int32)
    return lax.gather(vec, idx, _SPLAT_DNUMS, (1,),
                      mode=lax.GatherScatterMode.PROMISE_IN_BOUNDS)


def _tree8(v):
    a = [v[2 * i] + v[2 * i + 1] for i in range(4)]
    b = [a[0] + a[1], a[2] + a[3]]
    return b[0] + b[1]


def _sc_body(p_hbm, t_hbm, lab_hbm, out_hbm, pbuf, tbuf, labv, part,
             psem, tsem):
    c = lax.axis_index("c")
    s = lax.axis_index("s")
    wid = s * _NC + c
    base = wid * _RPW

    pltpu.sync_copy(lab_hbm.at[pl.ds(base, _RPW)], labv.at[pl.ds(0, _RPW)])

    zero = jnp.zeros((16,), jnp.float32)
    for i in range(1, _PPAD):
        part[pl.ds(i * 16, 16)] = zero

    iota = lax.iota(jnp.int32, 16)
    iota_gs = iota + 16
    iota_cnt = iota + (1 + _G) * 16
    ones = jnp.full((16,), 1.0, jnp.float32)

    def start_chunk(ci, b):
        rb = base + ci * _CHUNK
        hp = pltpu.async_copy(p_hbm.at[pl.ds(rb, _CHUNK)], pbuf.at[b], psem)
        ht = pltpu.async_copy(t_hbm.at[pl.ds(rb, _CHUNK)], tbuf.at[b], tsem)
        return hp, ht

    handles = [start_chunk(0, 0), start_chunk(1, 1)]

    def cnt_body(bi, carry):
        labvec = labv[pl.ds(bi * 16, 16)]
        plsc.addupdate_scatter(part, [labvec * 16 + iota_cnt], ones)
        return carry
    lax.fori_loop(0, _RPW // 16, cnt_body, 0)

    def compute_chunk(b, ci, acc_c):
        def blk_body(bi, acc_i, _b=b, _ci=ci):
            r0 = bi * _UNROLL
            labvec = labv[pl.ds(_ci * _CHUNK + r0, 16)]
            acc_l = list(acc_i)
            for r in range(_UNROLL):
                row = r0 + r
                pv = [pbuf[_b, row, pl.ds(k * 16, 16)] for k in range(8)]
                tv = [tbuf[_b, row, pl.ds(k * 16, 16)] for k in range(8)]
                for k in range(8):
                    dd = pv[k] - tv[k]
                    acc_l[k % 4] = acc_l[k % 4] + dd * dd
                rp = _tree8(pv)
                lab_splat = _splat(labvec, r)
                plsc.addupdate_scatter(part, [lab_splat * 16 + iota_gs], rp)
            return tuple(acc_l)
        return lax.fori_loop(0, _CHUNK // _UNROLL, blk_body, acc_c)

    acc = (zero, zero, zero, zero)

    def pair_body(pi, acc_c):
        ci0 = pi * 2
        handles[0][0].wait()
        handles[0][1].wait()
        acc_c = compute_chunk(0, ci0, acc_c)

        @pl.when(ci0 + 2 < _NCHUNK)
        def _():
            start_chunk(ci0 + 2, 0)

        handles[1][0].wait()
        handles[1][1].wait()
        acc_c = compute_chunk(1, ci0 + 1, acc_c)

        @pl.when(ci0 + 3 < _NCHUNK)
        def _():
            start_chunk(ci0 + 3, 1)
        return acc_c

    acc = lax.fori_loop(0, _NCHUNK // 2, pair_body, acc)

    part[pl.ds(0, 16)] = (acc[0] + acc[1]) + (acc[2] + acc[3])
    pltpu.sync_copy(part, out_hbm.at[wid])


_NBUF_TC = 11
_TC_NCH = _TC_ROWS // _TCB
_LR = _TCB // 128
_YROWS = 8 + 2 * _G * _LR


def _tc_body(p_hbm, t_hbm, lab_hbm, o_ref, pbufs, tbufs, labv,
             psems, tsems, lsem):
    lab_cp = pltpu.make_async_copy(
        lab_hbm.at[pl.ds(_TC_OFF, _TC_NCH)], labv, lsem)
    lab_cp.start()

    def issue(c, b):
        r = _SC_ROWS + c * _TCB
        pltpu.make_async_copy(
            p_hbm.at[pl.ds(r, _TCB), :], pbufs.at[b], psems.at[b]).start()
        pltpu.make_async_copy(
            t_hbm.at[pl.ds(r, _TCB), :], tbufs.at[b], tsems.at[b]).start()

    def waitfor(c, b):
        r = _SC_ROWS + c * _TCB
        pltpu.make_async_copy(
            p_hbm.at[pl.ds(r, _TCB), :], pbufs.at[b], psems.at[b]).wait()
        pltpu.make_async_copy(
            t_hbm.at[pl.ds(r, _TCB), :], tbufs.at[b], tsems.at[b]).wait()

    for b in range(_NBUF_TC):
        issue(b, b)
    lab_cp.wait()

    zsq = jnp.zeros((8, _D), jnp.float32)
    z2 = jnp.zeros((_LR, _D), jnp.float32)
    acc0 = (zsq,) + tuple(z2 for _ in range(2 * _G))

    def grp_body(g, acc):
        for b in range(_NBUF_TC):
            c = g * _NBUF_TC + b
            waitfor(c, b)
            p = pbufs[b]
            t = tbufs[b]
            d = p - t
            d2 = (d * d).reshape(_TCB // 8, 8, _D)
            sq = acc[0] + jnp.sum(d2, axis=0)
            rs2 = jnp.sum(p.reshape(_TCB // 128, 128, _D), axis=2)
            lab2 = labv[c]
            gl = []
            cl = []
            for gi in range(_G):
                m = lab2 == gi
                gl.append(acc[1 + gi] + jnp.where(m, rs2, 0.0))
                cl.append(acc[1 + _G + gi] + jnp.where(m, 1.0, 0.0))
            acc = (sq,) + tuple(gl) + tuple(cl)

            @pl.when(c + _NBUF_TC < _TC_NCH)
            def _(_c=c, _b=b):
                issue(_c + _NBUF_TC, _b)
        return acc

    acc = lax.fori_loop(0, _TC_NCH // _NBUF_TC, grp_body, acc0)

    def tail_chunk(c, b, acc):
        waitfor(c, b)
        p = pbufs[b]
        t = tbufs[b]
        d = p - t
        d2 = (d * d).reshape(_TCB // 8, 8, _D)
        sq = acc[0] + jnp.sum(d2, axis=0)
        rs2 = jnp.sum(p.reshape(_TCB // 128, 128, _D), axis=2)
        lab2 = labv[c]
        gl = []
        cl = []
        for gi in range(_G):
            m = lab2 == gi
            gl.append(acc[1 + gi] + jnp.where(m, rs2, 0.0))
            cl.append(acc[1 + _G + gi] + jnp.where(m, 1.0, 0.0))
        return (sq,) + tuple(gl) + tuple(cl)

    for b in range(_TC_NCH % _NBUF_TC):
        acc = tail_chunk((_TC_NCH // _NBUF_TC) * _NBUF_TC + b, b, acc)

    o_ref[pl.ds(0, 8), :] = acc[0]
    for gi in range(_G):
        o_ref[pl.ds(8 + _LR * gi, _LR), :] = acc[1 + gi]
        o_ref[pl.ds(8 + (_G + gi) * _LR, _LR), :] = acc[1 + _G + gi]


@jax.jit
def _sc_partials(predictions, targets, labels):
    mesh = plsc.VectorSubcoreMesh(core_axis_name="c", subcore_axis_name="s")
    f = functools.partial(
        pl.kernel,
        out_type=jax.ShapeDtypeStruct((_NW, _PPAD * 16), jnp.float32),
        mesh=mesh,
        compiler_params=pltpu.CompilerParams(needs_layout_passes=False),
        scratch_types=[
            pltpu.VMEM((2, _CHUNK, _D), jnp.float32),
            pltpu.VMEM((2, _CHUNK, _D), jnp.float32),
            pltpu.VMEM((_RPW + 16,), jnp.int32),
            pltpu.VMEM((_PPAD * 16,), jnp.float32),
            pltpu.SemaphoreType.DMA,
            pltpu.SemaphoreType.DMA,
        ],
    )(_sc_body)
    return f(predictions, targets, labels)


_TC_OFF = _SC_ROWS // _TCB


@jax.jit
def _tc_partials(p_full, t_full, lab_full):
    return pl.pallas_call(
        _tc_body,
        in_specs=[
            pl.BlockSpec(memory_space=pltpu.MemorySpace.HBM),
            pl.BlockSpec(memory_space=pltpu.MemorySpace.HBM),
            pl.BlockSpec(memory_space=pltpu.MemorySpace.HBM),
        ],
        out_specs=pl.BlockSpec(memory_space=pltpu.MemorySpace.VMEM),
        out_shape=jax.ShapeDtypeStruct((_YROWS, _D), jnp.float32),
        scratch_shapes=[
            pltpu.VMEM((_NBUF_TC, _TCB, _D), jnp.float32),
            pltpu.VMEM((_NBUF_TC, _TCB, _D), jnp.float32),
            pltpu.VMEM((_TC_NCH, _TCB // 128, 128), jnp.int32),
            pltpu.SemaphoreType.DMA((_NBUF_TC,)),
            pltpu.SemaphoreType.DMA((_NBUF_TC,)),
            pltpu.SemaphoreType.DMA,
        ],
    )(p_full, t_full, lab_full)


def _fin_body(x_ref, y_ref, o_ref):
    t0 = jnp.sum(x_ref[...], axis=0).reshape(1, 24 * 16)
    i384 = lax.broadcasted_iota(jnp.int32, (24 * 16, 24), 0)
    k384 = lax.broadcasted_iota(jnp.int32, (24 * 16, 24), 1)
    selx = jnp.where(i384 // 16 == k384, 1.0, 0.0)
    rx = lax.dot(t0, selx, precision=lax.Precision.HIGHEST,
                 preferred_element_type=jnp.float32)

    ys = y_ref[...]
    ry = lax.dot(ys, jnp.ones((_D, 1), jnp.float32),
                 precision=lax.Precision.HIGHEST,
                 preferred_element_type=jnp.float32)
    r40 = lax.broadcasted_iota(jnp.int32, (24, _YROWS), 1)
    k40 = lax.broadcasted_iota(jnp.int32, (24, _YROWS), 0)
    mid = 8 + _G * _LR
    reg = jnp.where(r40 < 8, 0,
                    jnp.where(r40 < mid, 1 + (r40 - 8) // _LR,
                              1 + _G + (r40 - mid) // _LR))
    sely = jnp.where(reg == k40, 1.0, 0.0)
    ryc = lax.dot(sely, ry, precision=lax.Precision.HIGHEST,
                  preferred_element_type=jnp.float32)

    n = float(_ROWS * _D)
    sq = rx[0, 0] + ryc[0, 0]
    gms = []
    for g in range(_G):
        gsum = rx[0, 1 + g] + ryc[1 + g, 0]
        gcnt = rx[0, 1 + _G + g] + ryc[1 + _G + g, 0]
        gms.append(gsum / (gcnt * _D))
    mm = sum(gms) / _G
    pen = sum((gm - mm) ** 2 for gm in gms) / (_G - 1)
    o_ref[0] = sq / n + pen


@jax.jit
def _finish(parts, tc):
    return pl.pallas_call(
        _fin_body,
        out_specs=pl.BlockSpec(memory_space=pltpu.MemorySpace.SMEM),
        out_shape=jax.ShapeDtypeStruct((1,), jnp.float32),
    )(parts, tc)


def kernel(predictions, targets, group_labels):
    labels = group_labels.astype(jnp.int32)
    parts = _sc_partials(predictions, targets, labels)
    lab3 = labels.reshape(_ROWS // _TCB, _TCB // 128, 128)
    tc = _tc_partials(predictions, targets, lab3)
    return _finish(parts, tc)[0]


# speedup vs baseline: 1.0056x; 1.0056x over previous
import functools

import jax
import jax.numpy as jnp
from jax import lax
from jax.experimental import pallas as pl
from jax.experimental.pallas import tpu as pltpu
from jax.experimental.pallas import tpu_sc as plsc

_G = 8
_ROWS = 16384
_D = 128
_NC = 2
_NS = 16
_NW = _NC * _NS
_SC_ROWS = 5120
_RPW = _SC_ROWS // _NW
_CHUNK = 40
_NCHUNK = _RPW // _CHUNK
_PR = 2 * _G + 1
_PPAD = 24
_UNROLL = 8
_TCB = 256
_TC_ROWS = _ROWS - _SC_ROWS

_SPLAT_DNUMS = lax.GatherDimensionNumbers(
    offset_dims=(), collapsed_slice_dims=(0,), start_index_map=(0,))


def _splat(vec, r):
    idx = jnp.full((16, 1), r, jnp.int32)
    return lax.gather(vec, idx, _SPLAT_DNUMS, (1,),
                      mode=lax.GatherScatterMode.PROMISE_IN_BOUNDS)


def _tree8(v):
    a = [v[2 * i] + v[2 * i + 1] for i in range(4)]
    b = [a[0] + a[1], a[2] + a[3]]
    return b[0] + b[1]


def _sc_body(p_hbm, t_hbm, lab_hbm, out_hbm, pbuf, tbuf, labv, part,
             psem, tsem):
    c = lax.axis_index("c")
    s = lax.axis_index("s")
    wid = s * _NC + c
    base = wid * _RPW

    pltpu.sync_copy(lab_hbm.at[pl.ds(base, _RPW)], labv.at[pl.ds(0, _RPW)])

    zero = jnp.zeros((16,), jnp.float32)
    for i in range(1, _PPAD):
        part[pl.ds(i * 16, 16)] = zero

    iota = lax.iota(jnp.int32, 16)
    iota_gs = iota + 16
    iota_cnt = iota + (1 + _G) * 16
    ones = jnp.full((16,), 1.0, jnp.float32)

    def start_chunk(ci, b):
        rb = base + ci * _CHUNK
        hp = pltpu.async_copy(p_hbm.at[pl.ds(rb, _CHUNK)], pbuf.at[b], psem)
        ht = pltpu.async_copy(t_hbm.at[pl.ds(rb, _CHUNK)], tbuf.at[b], tsem)
        return hp, ht

    handles = [start_chunk(0, 0), start_chunk(1, 1)]

    def cnt_body(bi, carry):
        labvec = labv[pl.ds(bi * 16, 16)]
        plsc.addupdate_scatter(part, [labvec * 16 + iota_cnt], ones)
        return carry
    lax.fori_loop(0, _RPW // 16, cnt_body, 0)

    def compute_chunk(b, ci, acc_c):
        def blk_body(bi, acc_i, _b=b, _ci=ci):
            r0 = bi * _UNROLL
            labvec = labv[pl.ds(_ci * _CHUNK + r0, 16)]
            acc_l = list(acc_i)
            for r in range(_UNROLL):
                row = r0 + r
                pv = [pbuf[_b, row, pl.ds(k * 16, 16)] for k in range(8)]
                tv = [tbuf[_b, row, pl.ds(k * 16, 16)] for k in range(8)]
                for k in range(8):
                    dd = pv[k] - tv[k]
                    acc_l[k % 4] = acc_l[k % 4] + dd * dd
                rp = _tree8(pv)
                lab_splat = _splat(labvec, r)
                plsc.addupdate_scatter(part, [lab_splat * 16 + iota_gs], rp)
            return tuple(acc_l)
        return lax.fori_loop(0, _CHUNK // _UNROLL, blk_body, acc_c)

    acc = (zero, zero, zero, zero)

    def pair_body(pi, acc_c):
        ci0 = pi * 2
        handles[0][0].wait()
        handles[0][1].wait()
        acc_c = compute_chunk(0, ci0, acc_c)

        @pl.when(ci0 + 2 < _NCHUNK)
        def _():
            start_chunk(ci0 + 2, 0)

        handles[1][0].wait()
        handles[1][1].wait()
        acc_c = compute_chunk(1, ci0 + 1, acc_c)

        @pl.when(ci0 + 3 < _NCHUNK)
        def _():
            start_chunk(ci0 + 3, 1)
        return acc_c

    acc = lax.fori_loop(0, _NCHUNK // 2, pair_body, acc)

    part[pl.ds(0, 16)] = (acc[0] + acc[1]) + (acc[2] + acc[3])
    pltpu.sync_copy(part, out_hbm.at[wid])


_NBUF_TC = 16
_TC_NCH = _TC_ROWS // _TCB
_LR = _TCB // 128
_YROWS = 8 + 2 * _G * _LR


def _tc_body(p_hbm, t_hbm, lab_hbm, o_ref, pbufs, tbufs, labv,
             psems, tsems, lsem):
    lab_cp = pltpu.make_async_copy(
        lab_hbm.at[pl.ds(_TC_OFF, _TC_NCH)], labv, lsem)
    lab_cp.start()

    def issue(c, b):
        r = _SC_ROWS + c * _TCB
        pltpu.make_async_copy(
            p_hbm.at[pl.ds(r, _TCB), :], pbufs.at[b], psems.at[b]).start()
        pltpu.make_async_copy(
            t_hbm.at[pl.ds(r, _TCB), :], tbufs.at[b], tsems.at[b]).start()

    def waitfor(c, b):
        r = _SC_ROWS + c * _TCB
        pltpu.make_async_copy(
            p_hbm.at[pl.ds(r, _TCB), :], pbufs.at[b], psems.at[b]).wait()
        pltpu.make_async_copy(
            t_hbm.at[pl.ds(r, _TCB), :], tbufs.at[b], tsems.at[b]).wait()

    for b in range(_NBUF_TC):
        issue(b, b)
    lab_cp.wait()

    zsq = jnp.zeros((8, _D), jnp.float32)
    z2 = jnp.zeros((_LR, _D), jnp.float32)
    acc0 = (zsq,) + tuple(z2 for _ in range(2 * _G))

    def grp_body(g, acc):
        for b in range(_NBUF_TC):
            c = g * _NBUF_TC + b
            waitfor(c, b)
            p = pbufs[b]
            t = tbufs[b]
            d = p - t
            d2 = (d * d).reshape(_TCB // 8, 8, _D)
            sq = acc[0] + jnp.sum(d2, axis=0)
            rs2 = jnp.sum(p.reshape(_TCB // 128, 128, _D), axis=2)
            lab2 = labv[c]
            gl = []
            cl = []
            for gi in range(_G):
                m = lab2 == gi
                gl.append(acc[1 + gi] + jnp.where(m, rs2, 0.0))
                cl.append(acc[1 + _G + gi] + jnp.where(m, 1.0, 0.0))
            acc = (sq,) + tuple(gl) + tuple(cl)

            @pl.when(c + _NBUF_TC < _TC_NCH)
            def _(_c=c, _b=b):
                issue(_c + _NBUF_TC, _b)
        return acc

    acc = lax.fori_loop(0, _TC_NCH // _NBUF_TC, grp_body, acc0)

    def tail_chunk(c, b, acc):
        waitfor(c, b)
        p = pbufs[b]
        t = tbufs[b]
        d = p - t
        d2 = (d * d).reshape(_TCB // 8, 8, _D)
        sq = acc[0] + jnp.sum(d2, axis=0)
        rs2 = jnp.sum(p.reshape(_TCB // 128, 128, _D), axis=2)
        lab2 = labv[c]
        gl = []
        cl = []
        for gi in range(_G):
            m = lab2 == gi
            gl.append(acc[1 + gi] + jnp.where(m, rs2, 0.0))
            cl.append(acc[1 + _G + gi] + jnp.where(m, 1.0, 0.0))
        return (sq,) + tuple(gl) + tuple(cl)

    for b in range(_TC_NCH % _NBUF_TC):
        acc = tail_chunk((_TC_NCH // _NBUF_TC) * _NBUF_TC + b, b, acc)

    o_ref[pl.ds(0, 8), :] = acc[0]
    for gi in range(_G):
        o_ref[pl.ds(8 + _LR * gi, _LR), :] = acc[1 + gi]
        o_ref[pl.ds(8 + (_G + gi) * _LR, _LR), :] = acc[1 + _G + gi]


@jax.jit
def _sc_partials(predictions, targets, labels):
    mesh = plsc.VectorSubcoreMesh(core_axis_name="c", subcore_axis_name="s")
    f = functools.partial(
        pl.kernel,
        out_type=jax.ShapeDtypeStruct((_NW, _PPAD * 16), jnp.float32),
        mesh=mesh,
        compiler_params=pltpu.CompilerParams(needs_layout_passes=False),
        scratch_types=[
            pltpu.VMEM((2, _CHUNK, _D), jnp.float32),
            pltpu.VMEM((2, _CHUNK, _D), jnp.float32),
            pltpu.VMEM((_RPW + 16,), jnp.int32),
            pltpu.VMEM((_PPAD * 16,), jnp.float32),
            pltpu.SemaphoreType.DMA,
            pltpu.SemaphoreType.DMA,
        ],
    )(_sc_body)
    return f(predictions, targets, labels)


_TC_OFF = _SC_ROWS // _TCB


@jax.jit
def _tc_partials(p_full, t_full, lab_full):
    return pl.pallas_call(
        _tc_body,
        in_specs=[
            pl.BlockSpec(memory_space=pltpu.MemorySpace.HBM),
            pl.BlockSpec(memory_space=pltpu.MemorySpace.HBM),
            pl.BlockSpec(memory_space=pltpu.MemorySpace.HBM),
        ],
        out_specs=pl.BlockSpec(memory_space=pltpu.MemorySpace.VMEM),
        out_shape=jax.ShapeDtypeStruct((_YROWS, _D), jnp.float32),
        scratch_shapes=[
            pltpu.VMEM((_NBUF_TC, _TCB, _D), jnp.float32),
            pltpu.VMEM((_NBUF_TC, _TCB, _D), jnp.float32),
            pltpu.VMEM((_TC_NCH, _TCB // 128, 128), jnp.int32),
            pltpu.SemaphoreType.DMA((_NBUF_TC,)),
            pltpu.SemaphoreType.DMA((_NBUF_TC,)),
            pltpu.SemaphoreType.DMA,
        ],
    )(p_full, t_full, lab_full)


def _fin_body(x_ref, y_ref, o_ref):
    t0 = jnp.sum(x_ref[...], axis=0).reshape(1, 24 * 16)
    i384 = lax.broadcasted_iota(jnp.int32, (24 * 16, 24), 0)
    k384 = lax.broadcasted_iota(jnp.int32, (24 * 16, 24), 1)
    selx = jnp.where(i384 // 16 == k384, 1.0, 0.0)
    rx = lax.dot(t0, selx, precision=lax.Precision.HIGHEST,
                 preferred_element_type=jnp.float32)

    ys = y_ref[...]
    ry = lax.dot(ys, jnp.ones((_D, 1), jnp.float32),
                 precision=lax.Precision.HIGHEST,
                 preferred_element_type=jnp.float32)
    r40 = lax.broadcasted_iota(jnp.int32, (24, _YROWS), 1)
    k40 = lax.broadcasted_iota(jnp.int32, (24, _YROWS), 0)
    mid = 8 + _G * _LR
    reg = jnp.where(r40 < 8, 0,
                    jnp.where(r40 < mid, 1 + (r40 - 8) // _LR,
                              1 + _G + (r40 - mid) // _LR))
    sely = jnp.where(reg == k40, 1.0, 0.0)
    ryc = lax.dot(sely, ry, precision=lax.Precision.HIGHEST,
                  preferred_element_type=jnp.float32)

    n = float(_ROWS * _D)
    sq = rx[0, 0] + ryc[0, 0]
    gms = []
    for g in range(_G):
        gsum = rx[0, 1 + g] + ryc[1 + g, 0]
        gcnt = rx[0, 1 + _G + g] + ryc[1 + _G + g, 0]
        gms.append(gsum / (gcnt * _D))
    mm = sum(gms) / _G
    pen = sum((gm - mm) ** 2 for gm in gms) / (_G - 1)
    o_ref[0] = sq / n + pen


@jax.jit
def _finish(parts, tc):
    return pl.pallas_call(
        _fin_body,
        out_specs=pl.BlockSpec(memory_space=pltpu.MemorySpace.SMEM),
        out_shape=jax.ShapeDtypeStruct((1,), jnp.float32),
    )(parts, tc)


def kernel(predictions, targets, group_labels):
    labels = group_labels.astype(jnp.int32)
    parts = _sc_partials(predictions, targets, labels)
    lab3 = labels.reshape(_ROWS // _TCB, _TCB // 128, 128)
    tc = _tc_partials(predictions, targets, lab3)
    return _finish(parts, tc)[0]


# speedup vs baseline: 1.0082x; 1.0026x over previous
import functools

import jax
import jax.numpy as jnp
from jax import lax
from jax.experimental import pallas as pl
from jax.experimental.pallas import tpu as pltpu
from jax.experimental.pallas import tpu_sc as plsc

_G = 8
_ROWS = 16384
_D = 128
_NC = 2
_NS = 16
_NW = _NC * _NS
_SC_ROWS = 5120
_RPW = _SC_ROWS // _NW
_CHUNK = 40
_NCHUNK = _RPW // _CHUNK
_PR = 2 * _G + 1
_PPAD = 24
_UNROLL = 8
_TCB = 256
_TC_ROWS = _ROWS - _SC_ROWS

_SPLAT_DNUMS = lax.GatherDimensionNumbers(
    offset_dims=(), collapsed_slice_dims=(0,), start_index_map=(0,))


def _splat(vec, r):
    idx = jnp.full((16, 1), r, jnp.int32)
    return lax.gather(vec, idx, _SPLAT_DNUMS, (1,),
                      mode=lax.GatherScatterMode.PROMISE_IN_BOUNDS)


def _tree8(v):
    a = [v[2 * i] + v[2 * i + 1] for i in range(4)]
    b = [a[0] + a[1], a[2] + a[3]]
    return b[0] + b[1]


def _sc_body(p_hbm, t_hbm, lab_hbm, out_hbm, pbuf, tbuf, labv, part,
             psem, tsem):
    c = lax.axis_index("c")
    s = lax.axis_index("s")
    wid = s * _NC + c
    base = wid * _RPW

    pltpu.sync_copy(lab_hbm.at[pl.ds(base, _RPW)], labv.at[pl.ds(0, _RPW)])

    zero = jnp.zeros((16,), jnp.float32)
    for i in range(1, _PPAD):
        part[pl.ds(i * 16, 16)] = zero

    iota = lax.iota(jnp.int32, 16)
    iota_gs = iota + 16
    iota_cnt = iota + (1 + _G) * 16
    ones = jnp.full((16,), 1.0, jnp.float32)

    def start_chunk(ci, b):
        rb = base + ci * _CHUNK
        hp = pltpu.async_copy(p_hbm.at[pl.ds(rb, _CHUNK)], pbuf.at[b], psem)
        ht = pltpu.async_copy(t_hbm.at[pl.ds(rb, _CHUNK)], tbuf.at[b], tsem)
        return hp, ht

    handles = [start_chunk(0, 0), start_chunk(1, 1)]

    def cnt_body(bi, carry):
        labvec = labv[pl.ds(bi * 16, 16)]
        plsc.addupdate_scatter(part, [labvec * 16 + iota_cnt], ones)
        return carry
    lax.fori_loop(0, _RPW // 16, cnt_body, 0)

    zero4 = (zero, zero, zero, zero)

    def chunk_body(ci, acc_c):
        b = lax.rem(ci, 2)
        handles[0][0].wait()
        handles[0][1].wait()

        def blk_body(bi, acc_i):
            r0 = bi * _UNROLL
            labvec = labv[pl.ds(ci * _CHUNK + r0, 16)]
            acc_l = list(acc_i)
            for r in range(_UNROLL):
                row = r0 + r
                pv = [pbuf[b, row, pl.ds(k * 16, 16)] for k in range(8)]
                tv = [tbuf[b, row, pl.ds(k * 16, 16)] for k in range(8)]
                for k in range(8):
                    dd = pv[k] - tv[k]
                    acc_l[k % 4] = acc_l[k % 4] + dd * dd
                rp = _tree8(pv)
                lab_splat = _splat(labvec, r)
                plsc.addupdate_scatter(part, [lab_splat * 16 + iota_gs], rp)
            return tuple(acc_l)

        acc_c = lax.fori_loop(0, _CHUNK // _UNROLL, blk_body, acc_c)

        @pl.when(ci + 2 < _NCHUNK)
        def _():
            start_chunk(ci + 2, b)
        return acc_c

    acc = lax.fori_loop(0, _NCHUNK, chunk_body, zero4)

    part[pl.ds(0, 16)] = (acc[0] + acc[1]) + (acc[2] + acc[3])
    pltpu.sync_copy(part, out_hbm.at[wid])


_NBUF_TC = 8
_TC_NCH = _TC_ROWS // _TCB
_LR = _TCB // 128
_YROWS = 8 + 2 * _G * _LR


def _tc_body(p_hbm, t_hbm, lab_hbm, o_ref, pbufs, tbufs, labv,
             psems, tsems, lsem):
    lab_cp = pltpu.make_async_copy(
        lab_hbm.at[pl.ds(_TC_OFF, _TC_NCH)], labv, lsem)
    lab_cp.start()

    def issue(c, b):
        r = _SC_ROWS + c * _TCB
        pltpu.make_async_copy(
            p_hbm.at[pl.ds(r, _TCB), :], pbufs.at[b], psems.at[b]).start()
        pltpu.make_async_copy(
            t_hbm.at[pl.ds(r, _TCB), :], tbufs.at[b], tsems.at[b]).start()

    def waitfor(c, b):
        r = _SC_ROWS + c * _TCB
        pltpu.make_async_copy(
            p_hbm.at[pl.ds(r, _TCB), :], pbufs.at[b], psems.at[b]).wait()
        pltpu.make_async_copy(
            t_hbm.at[pl.ds(r, _TCB), :], tbufs.at[b], tsems.at[b]).wait()

    for b in range(_NBUF_TC):
        issue(b, b)
    lab_cp.wait()

    zsq = jnp.zeros((8, _D), jnp.float32)
    z2 = jnp.zeros((_LR, _D), jnp.float32)
    acc0 = (zsq,) + tuple(z2 for _ in range(2 * _G))

    def grp_body(g, acc):
        for b in range(_NBUF_TC):
            c = g * _NBUF_TC + b
            waitfor(c, b)
            p = pbufs[b]
            t = tbufs[b]
            d = p - t
            d2 = (d * d).reshape(_TCB // 8, 8, _D)
            sq = acc[0] + jnp.sum(d2, axis=0)
            rs2 = jnp.sum(p.reshape(_TCB // 128, 128, _D), axis=2)
            lab2 = labv[c]
            gl = []
            cl = []
            for gi in range(_G):
                m = lab2 == gi
                gl.append(acc[1 + gi] + jnp.where(m, rs2, 0.0))
                cl.append(acc[1 + _G + gi] + jnp.where(m, 1.0, 0.0))
            acc = (sq,) + tuple(gl) + tuple(cl)

            @pl.when(c + _NBUF_TC < _TC_NCH)
            def _(_c=c, _b=b):
                issue(_c + _NBUF_TC, _b)
        return acc

    acc = lax.fori_loop(0, _TC_NCH // _NBUF_TC, grp_body, acc0)

    def tail_chunk(c, b, acc):
        waitfor(c, b)
        p = pbufs[b]
        t = tbufs[b]
        d = p - t
        d2 = (d * d).reshape(_TCB // 8, 8, _D)
        sq = acc[0] + jnp.sum(d2, axis=0)
        rs2 = jnp.sum(p.reshape(_TCB // 128, 128, _D), axis=2)
        lab2 = labv[c]
        gl = []
        cl = []
        for gi in range(_G):
            m = lab2 == gi
            gl.append(acc[1 + gi] + jnp.where(m, rs2, 0.0))
            cl.append(acc[1 + _G + gi] + jnp.where(m, 1.0, 0.0))
        return (sq,) + tuple(gl) + tuple(cl)

    for b in range(_TC_NCH % _NBUF_TC):
        acc = tail_chunk((_TC_NCH // _NBUF_TC) * _NBUF_TC + b, b, acc)

    o_ref[pl.ds(0, 8), :] = acc[0]
    for gi in range(_G):
        o_ref[pl.ds(8 + _LR * gi, _LR), :] = acc[1 + gi]
        o_ref[pl.ds(8 + (_G + gi) * _LR, _LR), :] = acc[1 + _G + gi]


@jax.jit
def _sc_partials(predictions, targets, labels):
    mesh = plsc.VectorSubcoreMesh(core_axis_name="c", subcore_axis_name="s")
    f = functools.partial(
        pl.kernel,
        out_type=jax.ShapeDtypeStruct((_NW, _PPAD * 16), jnp.float32),
        mesh=mesh,
        compiler_params=pltpu.CompilerParams(needs_layout_passes=False),
        scratch_types=[
            pltpu.VMEM((2, _CHUNK, _D), jnp.float32),
            pltpu.VMEM((2, _CHUNK, _D), jnp.float32),
            pltpu.VMEM((_RPW + 16,), jnp.int32),
            pltpu.VMEM((_PPAD * 16,), jnp.float32),
            pltpu.SemaphoreType.DMA,
            pltpu.SemaphoreType.DMA,
        ],
    )(_sc_body)
    return f(predictions, targets, labels)


_TC_OFF = _SC_ROWS // _TCB


@jax.jit
def _tc_partials(p_full, t_full, lab_full):
    return pl.pallas_call(
        _tc_body,
        in_specs=[
            pl.BlockSpec(memory_space=pltpu.MemorySpace.HBM),
            pl.BlockSpec(memory_space=pltpu.MemorySpace.HBM),
            pl.BlockSpec(memory_space=pltpu.MemorySpace.HBM),
        ],
        out_specs=pl.BlockSpec(memory_space=pltpu.MemorySpace.VMEM),
        out_shape=jax.ShapeDtypeStruct((_YROWS, _D), jnp.float32),
        scratch_shapes=[
            pltpu.VMEM((_NBUF_TC, _TCB, _D), jnp.float32),
            pltpu.VMEM((_NBUF_TC, _TCB, _D), jnp.float32),
            pltpu.VMEM((_TC_NCH, _TCB // 128, 128), jnp.int32),
            pltpu.SemaphoreType.DMA((_NBUF_TC,)),
            pltpu.SemaphoreType.DMA((_NBUF_TC,)),
            pltpu.SemaphoreType.DMA,
        ],
    )(p_full, t_full, lab_full)


def _fin_body(x_ref, y_ref, o_ref):
    t0 = jnp.sum(x_ref[...], axis=0).reshape(1, 24 * 16)
    i384 = lax.broadcasted_iota(jnp.int32, (24 * 16, 24), 0)
    k384 = lax.broadcasted_iota(jnp.int32, (24 * 16, 24), 1)
    selx = jnp.where(i384 // 16 == k384, 1.0, 0.0)
    rx = lax.dot(t0, selx, precision=lax.Precision.HIGHEST,
                 preferred_element_type=jnp.float32)

    ys = y_ref[...]
    ry = lax.dot(ys, jnp.ones((_D, 1), jnp.float32),
                 precision=lax.Precision.HIGHEST,
                 preferred_element_type=jnp.float32)
    r40 = lax.broadcasted_iota(jnp.int32, (24, _YROWS), 1)
    k40 = lax.broadcasted_iota(jnp.int32, (24, _YROWS), 0)
    mid = 8 + _G * _LR
    reg = jnp.where(r40 < 8, 0,
                    jnp.where(r40 < mid, 1 + (r40 - 8) // _LR,
                              1 + _G + (r40 - mid) // _LR))
    sely = jnp.where(reg == k40, 1.0, 0.0)
    ryc = lax.dot(sely, ry, precision=lax.Precision.HIGHEST,
                  preferred_element_type=jnp.float32)

    n = float(_ROWS * _D)
    sq = rx[0, 0] + ryc[0, 0]
    gms = []
    for g in range(_G):
        gsum = rx[0, 1 + g] + ryc[1 + g, 0]
        gcnt = rx[0, 1 + _G + g] + ryc[1 + _G + g, 0]
        gms.append(gsum / (gcnt * _D))
    mm = sum(gms) / _G
    pen = sum((gm - mm) ** 2 for gm in gms) / (_G - 1)
    o_ref[0] = sq / n + pen


@jax.jit
def _finish(parts, tc):
    return pl.pallas_call(
        _fin_body,
        out_specs=pl.BlockSpec(memory_space=pltpu.MemorySpace.SMEM),
        out_shape=jax.ShapeDtypeStruct((1,), jnp.float32),
    )(parts, tc)


def kernel(predictions, targets, group_labels):
    labels = group_labels.astype(jnp.int32)
    parts = _sc_partials(predictions, targets, labels)
    lab3 = labels.reshape(_ROWS // _TCB, _TCB // 128, 128)
    tc = _tc_partials(predictions, targets, lab3)
    return _finish(parts, tc)[0]


# speedup vs baseline: 1.0196x; 1.0113x over previous
import functools

import jax
import jax.numpy as jnp
from jax import lax
from jax.experimental import pallas as pl
from jax.experimental.pallas import tpu as pltpu
from jax.experimental.pallas import tpu_sc as plsc

_G = 8
_ROWS = 16384
_D = 128
_NC = 2
_NS = 16
_NW = _NC * _NS
_SC_ROWS = 5120
_RPW = _SC_ROWS // _NW
_CHUNK = 40
_NCHUNK = _RPW // _CHUNK
_PR = 2 * _G + 1
_PPAD = 24
_UNROLL = 8
_TCB = 256
_TC_ROWS = _ROWS - _SC_ROWS

_SPLAT_DNUMS = lax.GatherDimensionNumbers(
    offset_dims=(), collapsed_slice_dims=(0,), start_index_map=(0,))


def _splat(vec, r):
    idx = jnp.full((16, 1), r, jnp.int32)
    return lax.gather(vec, idx, _SPLAT_DNUMS, (1,),
                      mode=lax.GatherScatterMode.PROMISE_IN_BOUNDS)


def _tree8(v):
    a = [v[2 * i] + v[2 * i + 1] for i in range(4)]
    b = [a[0] + a[1], a[2] + a[3]]
    return b[0] + b[1]


def _sc_body(p_hbm, t_hbm, lab_hbm, out_hbm, pbuf, tbuf, labv, part,
             psem, tsem):
    c = lax.axis_index("c")
    s = lax.axis_index("s")
    wid = s * _NC + c
    base = wid * _RPW

    def start_chunk(ci, b):
        rb = base + ci * _CHUNK
        hp = pltpu.async_copy(p_hbm.at[pl.ds(rb, _CHUNK)], pbuf.at[b], psem)
        ht = pltpu.async_copy(t_hbm.at[pl.ds(rb, _CHUNK)], tbuf.at[b], tsem)
        return hp, ht

    handles = [start_chunk(0, 0), start_chunk(1, 1)]

    pltpu.sync_copy(lab_hbm.at[pl.ds(base, _RPW)], labv.at[pl.ds(0, _RPW)])

    zero = jnp.zeros((16,), jnp.float32)
    for i in range(1, _PPAD):
        part[pl.ds(i * 16, 16)] = zero

    iota = lax.iota(jnp.int32, 16)
    iota_gs = iota + 16
    iota_cnt = iota + (1 + _G) * 16
    ones = jnp.full((16,), 1.0, jnp.float32)

    def cnt_body(bi, carry):
        labvec = labv[pl.ds(bi * 16, 16)]
        plsc.addupdate_scatter(part, [labvec * 16 + iota_cnt], ones)
        return carry
    lax.fori_loop(0, _RPW // 16, cnt_body, 0)

    zero4 = (zero, zero, zero, zero)

    def chunk_body(ci, acc_c):
        b = lax.rem(ci, 2)
        handles[0][0].wait()
        handles[0][1].wait()

        def blk_body(bi, acc_i):
            r0 = bi * _UNROLL
            labvec = labv[pl.ds(ci * _CHUNK + r0, 16)]
            acc_l = list(acc_i)
            for r in range(_UNROLL):
                row = r0 + r
                pv = [pbuf[b, row, pl.ds(k * 16, 16)] for k in range(8)]
                tv = [tbuf[b, row, pl.ds(k * 16, 16)] for k in range(8)]
                for k in range(8):
                    dd = pv[k] - tv[k]
                    acc_l[k % 4] = acc_l[k % 4] + dd * dd
                rp = _tree8(pv)
                lab_splat = _splat(labvec, r)
                plsc.addupdate_scatter(part, [lab_splat * 16 + iota_gs], rp)
            return tuple(acc_l)

        acc_c = lax.fori_loop(0, _CHUNK // _UNROLL, blk_body, acc_c)

        @pl.when(ci + 2 < _NCHUNK)
        def _():
            start_chunk(ci + 2, b)
        return acc_c

    acc = lax.fori_loop(0, _NCHUNK, chunk_body, zero4)

    part[pl.ds(0, 16)] = (acc[0] + acc[1]) + (acc[2] + acc[3])
    pltpu.sync_copy(part, out_hbm.at[wid])


_NBUF_TC = 8
_TC_NCH = _TC_ROWS // _TCB
_LR = _TCB // 128
_YROWS = 8 + 2 * _G * _LR


def _tc_body(p_hbm, t_hbm, lab_hbm, o_ref, pbufs, tbufs, labv,
             psems, tsems, lsem):
    lab_cp = pltpu.make_async_copy(
        lab_hbm.at[pl.ds(_TC_OFF, _TC_NCH)], labv, lsem)
    lab_cp.start()

    def issue(c, b):
        r = _SC_ROWS + c * _TCB
        pltpu.make_async_copy(
            p_hbm.at[pl.ds(r, _TCB), :], pbufs.at[b], psems.at[b]).start()
        pltpu.make_async_copy(
            t_hbm.at[pl.ds(r, _TCB), :], tbufs.at[b], tsems.at[b]).start()

    def waitfor(c, b):
        r = _SC_ROWS + c * _TCB
        pltpu.make_async_copy(
            p_hbm.at[pl.ds(r, _TCB), :], pbufs.at[b], psems.at[b]).wait()
        pltpu.make_async_copy(
            t_hbm.at[pl.ds(r, _TCB), :], tbufs.at[b], tsems.at[b]).wait()

    for b in range(_NBUF_TC):
        issue(b, b)
    lab_cp.wait()

    zsq = jnp.zeros((8, _D), jnp.float32)
    z2 = jnp.zeros((_LR, _D), jnp.float32)
    acc0 = (zsq,) + tuple(z2 for _ in range(2 * _G))

    def grp_body(g, acc):
        for b in range(_NBUF_TC):
            c = g * _NBUF_TC + b
            waitfor(c, b)
            p = pbufs[b]
            t = tbufs[b]
            d = p - t
            d2 = (d * d).reshape(_TCB // 8, 8, _D)
            sq = acc[0] + jnp.sum(d2, axis=0)
            rs2 = jnp.sum(p.reshape(_TCB // 128, 128, _D), axis=2)
            lab2 = labv[c]
            gl = []
            cl = []
            for gi in range(_G):
                m = lab2 == gi
                gl.append(acc[1 + gi] + jnp.where(m, rs2, 0.0))
                cl.append(acc[1 + _G + gi] + jnp.where(m, 1.0, 0.0))
            acc = (sq,) + tuple(gl) + tuple(cl)

            @pl.when(c + _NBUF_TC < _TC_NCH)
            def _(_c=c, _b=b):
                issue(_c + _NBUF_TC, _b)
        return acc

    acc = lax.fori_loop(0, _TC_NCH // _NBUF_TC, grp_body, acc0)

    def tail_chunk(c, b, acc):
        waitfor(c, b)
        p = pbufs[b]
        t = tbufs[b]
        d = p - t
        d2 = (d * d).reshape(_TCB // 8, 8, _D)
        sq = acc[0] + jnp.sum(d2, axis=0)
        rs2 = jnp.sum(p.reshape(_TCB // 128, 128, _D), axis=2)
        lab2 = labv[c]
        gl = []
        cl = []
        for gi in range(_G):
            m = lab2 == gi
            gl.append(acc[1 + gi] + jnp.where(m, rs2, 0.0))
            cl.append(acc[1 + _G + gi] + jnp.where(m, 1.0, 0.0))
        return (sq,) + tuple(gl) + tuple(cl)

    for b in range(_TC_NCH % _NBUF_TC):
        acc = tail_chunk((_TC_NCH // _NBUF_TC) * _NBUF_TC + b, b, acc)

    o_ref[pl.ds(0, 8), :] = acc[0]
    for gi in range(_G):
        o_ref[pl.ds(8 + _LR * gi, _LR), :] = acc[1 + gi]
        o_ref[pl.ds(8 + (_G + gi) * _LR, _LR), :] = acc[1 + _G + gi]


@jax.jit
def _sc_partials(predictions, targets, labels):
    mesh = plsc.VectorSubcoreMesh(core_axis_name="c", subcore_axis_name="s")
    f = functools.partial(
        pl.kernel,
        out_type=jax.ShapeDtypeStruct((_NW, _PPAD * 16), jnp.float32),
        mesh=mesh,
        compiler_params=pltpu.CompilerParams(needs_layout_passes=False),
        scratch_types=[
            pltpu.VMEM((2, _CHUNK, _D), jnp.float32),
            pltpu.VMEM((2, _CHUNK, _D), jnp.float32),
            pltpu.VMEM((_RPW + 16,), jnp.int32),
            pltpu.VMEM((_PPAD * 16,), jnp.float32),
            pltpu.SemaphoreType.DMA,
            pltpu.SemaphoreType.DMA,
        ],
    )(_sc_body)
    return f(predictions, targets, labels)


_TC_OFF = _SC_ROWS // _TCB


@jax.jit
def _tc_partials(p_full, t_full, lab_full):
    return pl.pallas_call(
        _tc_body,
        in_specs=[
            pl.BlockSpec(memory_space=pltpu.MemorySpace.HBM),
            pl.BlockSpec(memory_space=pltpu.MemorySpace.HBM),
            pl.BlockSpec(memory_space=pltpu.MemorySpace.HBM),
        ],
        out_specs=pl.BlockSpec(memory_space=pltpu.MemorySpace.VMEM),
        out_shape=jax.ShapeDtypeStruct((_YROWS, _D), jnp.float32),
        scratch_shapes=[
            pltpu.VMEM((_NBUF_TC, _TCB, _D), jnp.float32),
            pltpu.VMEM((_NBUF_TC, _TCB, _D), jnp.float32),
            pltpu.VMEM((_TC_NCH, _TCB // 128, 128), jnp.int32),
            pltpu.SemaphoreType.DMA((_NBUF_TC,)),
            pltpu.SemaphoreType.DMA((_NBUF_TC,)),
            pltpu.SemaphoreType.DMA,
        ],
    )(p_full, t_full, lab_full)


def _fin_body(x_ref, y_ref, o_ref):
    t0 = jnp.sum(x_ref[...], axis=0).reshape(1, 24 * 16)
    i384 = lax.broadcasted_iota(jnp.int32, (24 * 16, 24), 0)
    k384 = lax.broadcasted_iota(jnp.int32, (24 * 16, 24), 1)
    selx = jnp.where(i384 // 16 == k384, 1.0, 0.0)
    rx = lax.dot(t0, selx, precision=lax.Precision.HIGHEST,
                 preferred_element_type=jnp.float32)

    ys = y_ref[...]
    ry = lax.dot(ys, jnp.ones((_D, 1), jnp.float32),
                 precision=lax.Precision.HIGHEST,
                 preferred_element_type=jnp.float32)
    r40 = lax.broadcasted_iota(jnp.int32, (24, _YROWS), 1)
    k40 = lax.broadcasted_iota(jnp.int32, (24, _YROWS), 0)
    mid = 8 + _G * _LR
    reg = jnp.where(r40 < 8, 0,
                    jnp.where(r40 < mid, 1 + (r40 - 8) // _LR,
                              1 + _G + (r40 - mid) // _LR))
    sely = jnp.where(reg == k40, 1.0, 0.0)
    ryc = lax.dot(sely, ry, precision=lax.Precision.HIGHEST,
                  preferred_element_type=jnp.float32)

    n = float(_ROWS * _D)
    sq = rx[0, 0] + ryc[0, 0]
    gms = []
    for g in range(_G):
        gsum = rx[0, 1 + g] + ryc[1 + g, 0]
        gcnt = rx[0, 1 + _G + g] + ryc[1 + _G + g, 0]
        gms.append(gsum / (gcnt * _D))
    mm = sum(gms) / _G
    pen = sum((gm - mm) ** 2 for gm in gms) / (_G - 1)
    o_ref[0] = sq / n + pen


@jax.jit
def _finish(parts, tc):
    return pl.pallas_call(
        _fin_body,
        out_specs=pl.BlockSpec(memory_space=pltpu.MemorySpace.SMEM),
        out_shape=jax.ShapeDtypeStruct((1,), jnp.float32),
    )(parts, tc)


def kernel(predictions, targets, group_labels):
    labels = group_labels.astype(jnp.int32)
    parts = _sc_partials(predictions, targets, labels)
    lab3 = labels.reshape(_ROWS // _TCB, _TCB // 128, 128)
    tc = _tc_partials(predictions, targets, lab3)
    return _finish(parts, tc)[0]
